# baseline (speedup 1.0000x reference)
import functools

import jax
import jax.numpy as jnp
from jax import lax
from jax.experimental import pallas as pl
from jax.experimental.pallas import tpu as pltpu
from jax.experimental.pallas import tpu_sc as plsc

B = 16384
F = 26
NC = 2
NS = 16
NW = NC * NS
BPW = B // NW
CHUNKS = BPW // 16
VOCAB_PAD = 1000448
GROUPS = [range(0, 7), range(7, 14), range(14, 20), range(20, 26)]


def _lr_kernel(x_hbm, t_hbm, bias_hbm, out_hbm, idx_v, vals_v, acc_v,
               bias_v, sem_i, sem_g):
    wid = lax.axis_index("s") * NC + lax.axis_index("c")
    base = wid * BPW

    pltpu.sync_copy(bias_hbm, bias_v)
    idx_copies = [
        pltpu.async_copy(
            x_hbm.at[f, pl.ds(base, BPW)],
            idx_v.at[pl.ds(f * BPW, BPW)],
            sem_i[g],
        )
        for g, fs in enumerate(GROUPS)
        for f in fs
    ]
    gathers = []
    k = 0
    for g, fs in enumerate(GROUPS):
        for _ in fs:
            idx_copies[k].wait()
            k += 1
        for f in fs:
            gathers.append(
                pltpu.async_copy(
                    t_hbm.at[0].at[idx_v.at[pl.ds(f * BPW, BPW)]],
                    vals_v.at[pl.ds(f * BPW, BPW)],
                    sem_g[g],
                ))

    bias_vec = bias_v[...]

    k = 0
    for g, fs in enumerate(GROUPS):
        for _ in fs:
            gathers[k].wait()
            k += 1

        def group_body(c, _, g=g, fs=fs):
            col = c * 16
            sl = pl.ds(col, 16)
            acc = bias_vec if g == 0 else acc_v[sl]
            for f in fs:
                acc = acc + vals_v[pl.ds(f * BPW + col, 16)]
            acc_v[sl] = acc
            return 0

        lax.fori_loop(0, CHUNKS, group_body, 0)

    pltpu.sync_copy(acc_v, out_hbm.at[pl.ds(base, BPW)])


@jax.jit
def _lr(x_t, t_flat, bias16):
    mesh = plsc.VectorSubcoreMesh(core_axis_name="c", subcore_axis_name="s",
                                  num_cores=NC)
    f = functools.partial(
        pl.kernel,
        out_type=jax.ShapeDtypeStruct((B,), jnp.float32),
        mesh=mesh,
        scratch_types=[
            pltpu.VMEM((F * BPW,), jnp.int32),
            pltpu.VMEM((F * BPW,), jnp.float32),
            pltpu.VMEM((BPW,), jnp.float32),
            pltpu.VMEM((16,), jnp.float32),
            [pltpu.SemaphoreType.DMA] * len(GROUPS),
            [pltpu.SemaphoreType.DMA] * len(GROUPS),
        ],
        compiler_params=pltpu.CompilerParams(needs_layout_passes=False,
                                             use_tc_tiling_on_sc=False),
    )(_lr_kernel)
    return f(x_t, t_flat, bias16)


def kernel(X, table, bias):
    x_t = X.T
    bias16 = jnp.broadcast_to(bias, (16,))
    out = _lr(x_t, table.T, bias16)
    return out.reshape(B, 1)


# speedup vs baseline: 1.6843x; 1.6843x over previous
import functools

import jax
import jax.numpy as jnp
from jax import lax
from jax.experimental import pallas as pl
from jax.experimental.pallas import tpu as pltpu
from jax.experimental.pallas import tpu_sc as plsc

B = 16384
F = 26
NC = 2
NS = 16
NW = NC * NS
BPW = B // NW
CHUNKS = BPW // 16
VOCAB_PAD = 1000448
GROUPS = [range(0, 7), range(7, 14), range(14, 20), range(20, 26)]


def _lr_kernel(x_hbm, t_hbm, bias_hbm, out_hbm, idx_v, vals_v, acc_v,
               bias_v, sem_i, sem_g):
    wid = lax.axis_index("s") * NC + lax.axis_index("c")
    base = wid * BPW

    pltpu.sync_copy(bias_hbm, bias_v)
    idx_copies = [
        pltpu.async_copy(
            x_hbm.at[f, pl.ds(base, BPW)],
            idx_v.at[pl.ds(f * BPW, BPW)],
            sem_i[g],
        )
        for g, fs in enumerate(GROUPS)
        for f in fs
    ]
    gathers = []
    k = 0
    for g, fs in enumerate(GROUPS):
        for _ in fs:
            idx_copies[k].wait()
            k += 1
        for f in fs:
            gathers.append(
                pltpu.async_copy(
                    t_hbm.at[idx_v.at[pl.ds(f * BPW, BPW)]],
                    vals_v.at[pl.ds(f * BPW, BPW)],
                    sem_g[g],
                ))

    bias_vec = bias_v[...]

    k = 0
    for g, fs in enumerate(GROUPS):
        for _ in fs:
            gathers[k].wait()
            k += 1

        def group_body(c, _, g=g, fs=fs):
            col = c * 16
            sl = pl.ds(col, 16)
            acc = bias_vec if g == 0 else acc_v[sl]
            for f in fs:
                acc = acc + vals_v[pl.ds(f * BPW + col, 16)]
            acc_v[sl] = acc
            return 0

        lax.fori_loop(0, CHUNKS, group_body, 0)

    pltpu.sync_copy(acc_v, out_hbm.at[pl.ds(base, BPW)])


@jax.jit
def _lr(x_t, t_flat, bias16):
    mesh = plsc.VectorSubcoreMesh(core_axis_name="c", subcore_axis_name="s",
                                  num_cores=NC)
    f = functools.partial(
        pl.kernel,
        out_type=jax.ShapeDtypeStruct((B,), jnp.float32),
        mesh=mesh,
        scratch_types=[
            pltpu.VMEM((F * BPW,), jnp.int32),
            pltpu.VMEM((F * BPW,), jnp.float32),
            pltpu.VMEM((BPW,), jnp.float32),
            pltpu.VMEM((16,), jnp.float32),
            [pltpu.SemaphoreType.DMA] * len(GROUPS),
            [pltpu.SemaphoreType.DMA] * len(GROUPS),
        ],
        compiler_params=pltpu.CompilerParams(needs_layout_passes=False),
    )(_lr_kernel)
    return f(x_t, t_flat, bias16)


def kernel(X, table, bias):
    x_t = X.T
    t_flat = jnp.pad(table, ((0, VOCAB_PAD - 1000000), (0, 0))).reshape(-1)
    bias16 = jnp.broadcast_to(bias, (16,))
    out = _lr(x_t, t_flat, bias16)
    return out.reshape(B, 1)


# speedup vs baseline: 1.7175x; 1.0197x over previous
import functools

import jax
import jax.numpy as jnp
from jax import lax
from jax.experimental import pallas as pl
from jax.experimental.pallas import tpu as pltpu
from jax.experimental.pallas import tpu_sc as plsc

B = 16384
F = 26
NC = 2
NS = 16
NW = NC * NS
BPW = B // NW
CHUNKS = BPW // 16
VOCAB_PAD = 1000448
GROUPS = [range(0, 4), range(4, 8), range(8, 11), range(11, 14),
          range(14, 17), range(17, 20), range(20, 23), range(23, 26)]


def _lr_kernel(x_hbm, t_hbm, bias_hbm, out_hbm, idx_v, vals_v, acc_v,
               bias_v, sem_i, sem_g):
    wid = lax.axis_index("s") * NC + lax.axis_index("c")
    base = wid * BPW

    pltpu.sync_copy(bias_hbm, bias_v)
    idx_copies = [
        pltpu.async_copy(
            x_hbm.at[f, pl.ds(base, BPW)],
            idx_v.at[pl.ds(f * BPW, BPW)],
            sem_i[g],
        )
        for g, fs in enumerate(GROUPS)
        for f in fs
    ]
    gathers = []
    k = 0
    for g, fs in enumerate(GROUPS):
        for _ in fs:
            idx_copies[k].wait()
            k += 1
        for f in fs:
            gathers.append(
                pltpu.async_copy(
                    t_hbm.at[idx_v.at[pl.ds(f * BPW, BPW)]],
                    vals_v.at[pl.ds(f * BPW, BPW)],
                    sem_g[g],
                ))

    bias_vec = bias_v[...]

    k = 0
    for g, fs in enumerate(GROUPS):
        for _ in fs:
            gathers[k].wait()
            k += 1

        def group_body(c, _, g=g, fs=fs):
            col = c * 16
            sl = pl.ds(col, 16)
            acc = bias_vec if g == 0 else acc_v[sl]
            for f in fs:
                acc = acc + vals_v[pl.ds(f * BPW + col, 16)]
            acc_v[sl] = acc
            return 0

        lax.fori_loop(0, CHUNKS, group_body, 0)

    pltpu.sync_copy(acc_v, out_hbm.at[pl.ds(base, BPW)])


@jax.jit
def _lr(x_t, t_flat, bias16):
    mesh = plsc.VectorSubcoreMesh(core_axis_name="c", subcore_axis_name="s",
                                  num_cores=NC)
    f = functools.partial(
        pl.kernel,
        out_type=jax.ShapeDtypeStruct((B,), jnp.float32),
        mesh=mesh,
        scratch_types=[
            pltpu.VMEM((F * BPW,), jnp.int32),
            pltpu.VMEM((F * BPW,), jnp.float32),
            pltpu.VMEM((BPW,), jnp.float32),
            pltpu.VMEM((16,), jnp.float32),
            [pltpu.SemaphoreType.DMA] * len(GROUPS),
            [pltpu.SemaphoreType.DMA] * len(GROUPS),
        ],
        compiler_params=pltpu.CompilerParams(needs_layout_passes=False),
    )(_lr_kernel)
    return f(x_t, t_flat, bias16)


def kernel(X, table, bias):
    x_t = X.T
    t_flat = jnp.pad(table, ((0, VOCAB_PAD - 1000000), (0, 0))).reshape(-1)
    bias16 = jnp.broadcast_to(bias, (16,))
    out = _lr(x_t, t_flat, bias16)
    return out.reshape(B, 1)
